# baseline (speedup 1.0000x reference)
import functools

import jax
import jax.numpy as jnp
from jax import lax
from jax.experimental import pallas as pl
from jax.experimental.pallas import tpu as pltpu
from jax.experimental.pallas import tpu_sc as plsc

LANES = 16


@functools.lru_cache(maxsize=None)
def _make_kernel(B: int, n_loc: int, n_grp: int):
    info = plsc.get_sparse_core_info()
    n_cores = 1
    n_sub = info.num_subcores
    nw = n_cores * n_sub
    bpw = B // nw
    assert B % (8 * nw) == 0 and bpw % LANES == 0
    mesh = plsc.VectorSubcoreMesh(
        core_axis_name="c", subcore_axis_name="s",
        num_cores=n_cores, num_subcores=n_sub)

    @functools.partial(
        pl.kernel,
        mesh=mesh,
        out_type=jax.ShapeDtypeStruct((B,), jnp.float32),
        compiler_params=pltpu.CompilerParams(
            needs_layout_passes=False,
            disable_bounds_checks=True,
            disable_semaphore_checks=True,
            skip_device_barrier=True,
            use_tc_tiling_on_sc=False,
        ),
        scratch_types=[
            pltpu.VMEM((bpw,), jnp.int32),
            pltpu.VMEM((n_loc,), jnp.int32),
            pltpu.VMEM((n_grp,), jnp.float32),
            pltpu.VMEM((n_loc,), jnp.float32),
            pltpu.VMEM((bpw,), jnp.float32),
            pltpu.SemaphoreType.DMA,
            pltpu.SemaphoreType.DMA,
        ],
    )
    def k(loc_hbm, lg_hbm, par_hbm, out_hbm, loc_v, lg_v, par_v, fused_v, out_v,
          sem_loc, sem_tab):
        wid = lax.axis_index("s") * n_cores + lax.axis_index("c")
        base = wid * bpw
        c_loc = pltpu.async_copy(loc_hbm.at[pl.ds(base, bpw)], loc_v, sem_loc)
        c_lg = pltpu.async_copy(lg_hbm, lg_v, sem_tab)
        c_par = pltpu.async_copy(par_hbm, par_v, sem_tab)
        c_lg.wait()
        c_par.wait()

        @plsc.parallel_loop(0, n_loc, LANES, unroll=8)
        def _(j):
            gv = lg_v[pl.ds(j, LANES)]
            fused_v[pl.ds(j, LANES)] = plsc.load_gather(par_v, [gv])

        c_loc.wait()

        @plsc.parallel_loop(0, bpw, LANES, unroll=16)
        def _(i):
            lv = loc_v[pl.ds(i, LANES)]
            out_v[pl.ds(i, LANES)] = plsc.load_gather(fused_v, [lv])

        pltpu.sync_copy(out_v, out_hbm.at[pl.ds(base, bpw)])

    return k


def kernel(locations, loc_group, params):
    out = _make_kernel(locations.shape[0], loc_group.shape[0], params.shape[0])(
        locations, loc_group, params)
    return out[:, None]


# speedup vs baseline: 1.0154x; 1.0154x over previous
import functools

import jax
import jax.numpy as jnp
from jax import lax
from jax.experimental import pallas as pl
from jax.experimental.pallas import tpu as pltpu
from jax.experimental.pallas import tpu_sc as plsc

LANES = 16


@functools.lru_cache(maxsize=None)
def _make_kernel(B: int, n_loc: int, n_grp: int):
    info = plsc.get_sparse_core_info()
    n_cores = 1
    n_sub = info.num_subcores
    nw = n_cores * n_sub
    bpw = B // nw
    assert B % (8 * nw) == 0 and bpw % LANES == 0
    mesh = plsc.VectorSubcoreMesh(
        core_axis_name="c", subcore_axis_name="s",
        num_cores=n_cores, num_subcores=n_sub)

    @functools.partial(
        pl.kernel,
        mesh=mesh,
        out_type=jax.ShapeDtypeStruct((B,), jnp.float32),
        compiler_params=pltpu.CompilerParams(
            needs_layout_passes=False,
            disable_bounds_checks=True,
            disable_semaphore_checks=True,
            skip_device_barrier=True,
            use_tc_tiling_on_sc=False,
        ),
        scratch_types=[
            pltpu.VMEM((bpw,), jnp.int32),
            pltpu.VMEM((n_loc,), jnp.int32),
            pltpu.VMEM((n_grp,), jnp.float32),
            pltpu.VMEM((n_loc,), jnp.float32),
            pltpu.VMEM((bpw,), jnp.float32),
            pltpu.SemaphoreType.DMA,
            pltpu.SemaphoreType.DMA,
        ],
    )
    def k(loc_hbm, lg_hbm, par_hbm, out_hbm, loc_v, lg_v, par_v, fused_v, out_v,
          sem_loc, sem_tab):
        wid = lax.axis_index("s") * n_cores + lax.axis_index("c")
        base = wid * bpw
        c_loc = pltpu.async_copy(loc_hbm.at[pl.ds(base, bpw)], loc_v, sem_loc)
        c_lg = pltpu.async_copy(lg_hbm, lg_v, sem_tab)
        c_par = pltpu.async_copy(par_hbm, par_v, sem_tab)
        c_lg.wait()
        c_par.wait()

        @plsc.parallel_loop(0, n_loc, LANES, unroll=8)
        def _(j):
            gv = lg_v[pl.ds(j, LANES)]
            fused_v[pl.ds(j, LANES)] = plsc.load_gather(par_v, [gv])

        c_loc.wait()

        @plsc.parallel_loop(0, bpw, LANES, unroll=4)
        def _(i):
            lv = loc_v[pl.ds(i, LANES)]
            out_v[pl.ds(i, LANES)] = plsc.load_gather(fused_v, [lv])

        pltpu.sync_copy(out_v, out_hbm.at[pl.ds(base, bpw)])

    return k


def kernel(locations, loc_group, params):
    out = _make_kernel(locations.shape[0], loc_group.shape[0], params.shape[0])(
        locations, loc_group, params)
    return out[:, None]
